# baseline (speedup 1.0000x reference)
import jax
import jax.numpy as jnp
from jax import lax
from jax.experimental import pallas as pl
from jax.experimental.pallas import tpu as pltpu
from jax.experimental.pallas import tpu_sc as plsc

_B, _S, _D = 16384, 50, 128
_N = _B * _S
_NC, _NS = 2, 16
_NW = _NC * _NS
_PER_W = _N // _NW
_CH = 128
_NCH = _PER_W // _CH
_NBUF = 4


def _emb_body(ids_hbm, table_hbm, out_hbm, idx_v,
              b0, b1, b2, b3, g0, g1, g2, g3, w0, w1, w2, w3):
    bufs = (b0, b1, b2, b3)
    gsems = (g0, g1, g2, g3)
    wsems = (w0, w1, w2, w3)
    wid = lax.axis_index("s") * _NC + lax.axis_index("c")
    row0 = wid * _PER_W

    pltpu.sync_copy(ids_hbm.at[wid], idx_v)

    def out_at(j):
        return out_hbm.at[pl.ds(row0 + j * _CH, _CH)]

    def start_gather(j, k):
        pltpu.async_copy(table_hbm.at[idx_v.at[j]], bufs[k], gsems[k])

    def wait_gather(j, k):
        pltpu.make_async_copy(table_hbm.at[idx_v.at[j]], bufs[k], gsems[k]).wait()

    def start_write(j, k):
        pltpu.async_copy(bufs[k], out_at(j), wsems[k])

    def wait_write(j, k):
        pltpu.make_async_copy(bufs[k], out_at(j), wsems[k]).wait()

    start_gather(0, 0)
    start_gather(1, 1)

    for j in (0, 1):
        wait_gather(j, j % _NBUF)
        start_write(j, j % _NBUF)
        start_gather(j + 2, (j + 2) % _NBUF)

    def body(i, carry):
        j0 = 4 * i + 2
        for r in range(4):
            j = j0 + r
            k = (2 + r) % _NBUF
            k2 = (2 + r + 2) % _NBUF
            wait_gather(j, k)
            start_write(j, k)
            wait_write(j - 2, k2)
            start_gather(j + 2, k2)
        return carry

    lax.fori_loop(0, (_NCH - 4) // 4, body, 0)

    for j in (_NCH - 2, _NCH - 1):
        wait_gather(j, j % _NBUF)
        start_write(j, j % _NBUF)

    for j in (_NCH - 4, _NCH - 3, _NCH - 2, _NCH - 1):
        wait_write(j, j % _NBUF)


@jax.jit
def kernel(token_ids, weight):
    ids = token_ids.reshape(_NW, _NCH, _CH).astype(jnp.int32)
    mesh = plsc.VectorSubcoreMesh(core_axis_name="c", subcore_axis_name="s")
    out = pl.kernel(
        _emb_body,
        mesh=mesh,
        out_type=jax.ShapeDtypeStruct((_N, _D), jnp.float32),
        scratch_types=[
            pltpu.VMEM((_NCH, _CH), jnp.int32),
            pltpu.VMEM((_CH, _D), jnp.float32),
            pltpu.VMEM((_CH, _D), jnp.float32),
            pltpu.VMEM((_CH, _D), jnp.float32),
            pltpu.VMEM((_CH, _D), jnp.float32),
            pltpu.SemaphoreType.DMA,
            pltpu.SemaphoreType.DMA,
            pltpu.SemaphoreType.DMA,
            pltpu.SemaphoreType.DMA,
            pltpu.SemaphoreType.DMA,
            pltpu.SemaphoreType.DMA,
            pltpu.SemaphoreType.DMA,
            pltpu.SemaphoreType.DMA,
        ],
    )(ids, weight)
    return out.reshape(_B, _S, _D)


# speedup vs baseline: 1.0026x; 1.0026x over previous
import jax
import jax.numpy as jnp
from jax import lax
from jax.experimental import pallas as pl
from jax.experimental.pallas import tpu as pltpu
from jax.experimental.pallas import tpu_sc as plsc

_B, _S, _D = 16384, 50, 128
_N = _B * _S
_NC, _NS = 2, 16
_NW = _NC * _NS
_PER_W = _N // _NW
_CH = 128
_NCH = _PER_W // _CH
_NBUF = 6
_W = _NBUF // 2


def _emb_body(ids_hbm, table_hbm, out_hbm, idx_v, *rest):
    bufs = rest[:_NBUF]
    gsems = rest[_NBUF:2 * _NBUF]
    wsems = rest[2 * _NBUF:]
    wid = lax.axis_index("s") * _NC + lax.axis_index("c")
    row0 = wid * _PER_W

    pltpu.sync_copy(ids_hbm.at[wid], idx_v)

    def out_at(j):
        return out_hbm.at[pl.ds(row0 + j * _CH, _CH)]

    def start_gather(j, k):
        pltpu.async_copy(table_hbm.at[idx_v.at[j]], bufs[k], gsems[k])

    def wait_gather(j, k):
        pltpu.make_async_copy(table_hbm.at[idx_v.at[j]], bufs[k], gsems[k]).wait()

    def start_write(j, k):
        pltpu.async_copy(bufs[k], out_at(j), wsems[k])

    def wait_write(j, k):
        pltpu.make_async_copy(bufs[k], out_at(j), wsems[k]).wait()

    def step(j, k, prefetch, wait_w):
        wait_gather(j, k)
        start_write(j, k)
        if prefetch:
            k2 = (k + _W) % _NBUF
            if wait_w:
                wait_write(j - _W, k2)
            start_gather(j + _W, k2)

    for j in range(_W):
        start_gather(j, j % _NBUF)

    for j in range(_W):
        step(j, j % _NBUF, prefetch=True, wait_w=False)

    n_steady = _NCH - 2 * _W
    n_groups = n_steady // _NBUF

    def body(i, carry):
        j0 = _NBUF * i + _W
        for r in range(_NBUF):
            step(j0 + r, (_W + r) % _NBUF, prefetch=True, wait_w=True)
        return carry

    lax.fori_loop(0, n_groups, body, 0)

    for j in range(_W + n_groups * _NBUF, _NCH - _W):
        step(j, j % _NBUF, prefetch=True, wait_w=True)

    for j in range(_NCH - _W, _NCH):
        step(j, j % _NBUF, prefetch=False, wait_w=False)

    for j in range(_NCH - _NBUF, _NCH):
        wait_write(j, j % _NBUF)


@jax.jit
def kernel(token_ids, weight):
    ids = token_ids.reshape(_NW, _NCH, _CH).astype(jnp.int32)
    mesh = plsc.VectorSubcoreMesh(core_axis_name="c", subcore_axis_name="s")
    out = pl.kernel(
        _emb_body,
        mesh=mesh,
        out_type=jax.ShapeDtypeStruct((_N, _D), jnp.float32),
        scratch_types=(
            [pltpu.VMEM((_NCH, _CH), jnp.int32)]
            + [pltpu.VMEM((_CH, _D), jnp.float32)] * _NBUF
            + [pltpu.SemaphoreType.DMA] * (2 * _NBUF)
        ),
    )(ids, weight)
    return out.reshape(_B, _S, _D)
